# Initial kernel scaffold; baseline (speedup 1.0000x reference)
import jax
import jax.numpy as jnp
from jax.experimental import pallas as pl


def kernel(batchinput_tensor, embs_A, W_ih0, W_hh0, b_ih0, b_hh0, W_ih1, W_hh1, b_ih1, b_hh1, W_global, b_global):
    raise NotImplementedError("write your pallas kernel here")


# speedup vs baseline: 2.0934x; 2.0934x over previous
import functools

import jax
import jax.numpy as jnp
from jax import lax
from jax.experimental import pallas as pl
from jax.experimental.pallas import tpu as pltpu
from jax.experimental.pallas import tpu_sc as plsc

B = 32
S = 32
DIM = 512
HID = 512
G4 = 4 * HID
N_ROWS = B * S
VOCAB = 10000


@functools.lru_cache(maxsize=1)
def _make_sc_gather():
    info = plsc.get_sparse_core_info()
    nc, ns = info.num_cores, info.num_subcores
    nw = nc * ns
    rows_per_w = N_ROWS // nw
    mesh = plsc.VectorSubcoreMesh(core_axis_name="c", subcore_axis_name="s")

    @functools.partial(
        pl.kernel,
        mesh=mesh,
        out_type=jax.ShapeDtypeStruct((N_ROWS, DIM), jnp.float32),
        scratch_types=[
            pltpu.VMEM((rows_per_w,), jnp.int32),
            pltpu.VMEM((rows_per_w, DIM), jnp.float32),
            pltpu.SemaphoreType.DMA,
        ],
    )
    def gather_k(idx_hbm, table_hbm, out_hbm, idx_v, rows_v, sem):
        wid = lax.axis_index("s") * nc + lax.axis_index("c")
        base = wid * rows_per_w
        pltpu.sync_copy(idx_hbm.at[pl.ds(base, rows_per_w)], idx_v)
        pltpu.async_copy(table_hbm.at[idx_v], rows_v, sem).wait()
        pltpu.sync_copy(rows_v, out_hbm.at[pl.ds(base, rows_per_w)])

    return gather_k


def _lstm_body(x_ref, wih0_ref, whh0_ref, wcat1_ref, b0_ref, b1_ref,
               y_ref, xi0_ref, hcat_ref, c1_ref, c2_ref):
    xi0_ref[...] = (
        jnp.dot(x_ref[...], wih0_ref[...], preferred_element_type=jnp.float32)
        + b0_ref[...]
    )
    hcat_ref[...] = jnp.zeros((B, 2 * HID), jnp.float32)
    c1_ref[...] = jnp.zeros((B, HID), jnp.float32)
    c2_ref[...] = jnp.zeros((B, HID), jnp.float32)

    def step(t, _):
        h1 = hcat_ref[:, :HID]
        g1 = xi0_ref[pl.ds(t * B, B), :] + jnp.dot(
            h1, whh0_ref[...], preferred_element_type=jnp.float32)
        i1 = jax.nn.sigmoid(g1[:, 0:HID])
        f1 = jax.nn.sigmoid(g1[:, HID:2 * HID])
        gg1 = jnp.tanh(g1[:, 2 * HID:3 * HID])
        o1 = jax.nn.sigmoid(g1[:, 3 * HID:4 * HID])
        c1 = f1 * c1_ref[...] + i1 * gg1
        c1_ref[...] = c1
        hcat_ref[:, :HID] = o1 * jnp.tanh(c1)

        g2 = jnp.dot(hcat_ref[...], wcat1_ref[...],
                     preferred_element_type=jnp.float32) + b1_ref[...]
        i2 = jax.nn.sigmoid(g2[:, 0:HID])
        f2 = jax.nn.sigmoid(g2[:, HID:2 * HID])
        gg2 = jnp.tanh(g2[:, 2 * HID:3 * HID])
        o2 = jax.nn.sigmoid(g2[:, 3 * HID:4 * HID])
        c2 = f2 * c2_ref[...] + i2 * gg2
        c2_ref[...] = c2
        h2 = o2 * jnp.tanh(c2)
        hcat_ref[:, HID:] = h2
        y_ref[pl.ds(t * B, B), :] = h2
        return 0

    lax.fori_loop(0, S, step, 0)


def _lstm(x, wih0T, whh0T, wcat1T, b0, b1):
    return pl.pallas_call(
        _lstm_body,
        out_shape=jax.ShapeDtypeStruct((N_ROWS, HID), jnp.float32),
        scratch_shapes=[
            pltpu.VMEM((N_ROWS, G4), jnp.float32),
            pltpu.VMEM((B, 2 * HID), jnp.float32),
            pltpu.VMEM((B, HID), jnp.float32),
            pltpu.VMEM((B, HID), jnp.float32),
        ],
    )(x, wih0T, whh0T, wcat1T, b0, b1)


_HEAD_TILE = 128


def _head_body(y_ref, wg_ref, bg_ref, out_ref):
    logits = jnp.dot(y_ref[...], wg_ref[...],
                     preferred_element_type=jnp.float32) + bg_ref[...]
    m = jnp.max(logits, axis=1, keepdims=True)
    lse = jnp.log(jnp.sum(jnp.exp(logits - m), axis=1, keepdims=True)) + m
    out_ref[...] = logits - lse


def _head(y, wgT, bg):
    n_tiles = N_ROWS // _HEAD_TILE
    return pl.pallas_call(
        _head_body,
        grid=(n_tiles,),
        in_specs=[
            pl.BlockSpec((_HEAD_TILE, HID), lambda i: (i, 0)),
            pl.BlockSpec((HID, VOCAB), lambda i: (0, 0)),
            pl.BlockSpec((1, VOCAB), lambda i: (0, 0)),
        ],
        out_specs=pl.BlockSpec((_HEAD_TILE, VOCAB), lambda i: (i, 0)),
        out_shape=jax.ShapeDtypeStruct((N_ROWS, VOCAB), jnp.float32),
    )(y, wgT, bg)


def kernel(batchinput_tensor, embs_A, W_ih0, W_hh0, b_ih0, b_hh0,
           W_ih1, W_hh1, b_ih1, b_hh1, W_global, b_global):
    idx_t = batchinput_tensor[:, :, 0].astype(jnp.int32).T.reshape(N_ROWS)
    x = _make_sc_gather()(idx_t, embs_A)

    wih0T = W_ih0.T
    whh0T = W_hh0.T
    wcat1T = jnp.concatenate([W_ih1.T, W_hh1.T], axis=0)
    b0 = (b_ih0 + b_hh0).reshape(1, G4)
    b1 = (b_ih1 + b_hh1).reshape(1, G4)
    y_t = _lstm(x, wih0T, whh0T, wcat1T, b0, b1)

    task1 = y_t.reshape(S, B, HID).transpose(1, 0, 2).reshape(N_ROWS, HID)
    out = _head(task1, W_global.T, b_global.reshape(1, VOCAB))
    return (out, jnp.zeros((N_ROWS,), dtype=jnp.int32))
